# baseline (speedup 1.0000x reference)
import functools
import jax
import jax.numpy as jnp
from jax import lax
from jax.experimental import pallas as pl
from jax.experimental.pallas import tpu as pltpu
from jax.experimental.pallas import tpu_sc as plsc

_NC = 2
_NS = 16
_L = 16
_C = 8


def _lane_gather(v, idx):
    dnums = lax.GatherDimensionNumbers(
        offset_dims=(), collapsed_slice_dims=(0,), start_index_map=(0,))
    return lax.gather(
        v, idx[:, None], dnums, slice_sizes=(1,),
        mode=lax.GatherScatterMode.PROMISE_IN_BOUNDS)


def _butterfly_sum(v):
    lanes = jnp.arange(_L, dtype=jnp.int32)
    for s in (1, 2, 4, 8):
        v = v + _lane_gather(v, lanes ^ s)
    return v


def _make_sc_attend(B, M, d, row0, nrows):
    nw = _NC * _NS
    rows_per_w = nrows // nw
    n_chunks = rows_per_w // _C
    nk = d // _L
    mesh = plsc.VectorSubcoreMesh(core_axis_name="c", subcore_axis_name="s")

    @functools.partial(
        pl.kernel,
        mesh=mesh,
        out_type=jax.ShapeDtypeStruct((nrows, d), jnp.float32),
        scratch_types=[
            pltpu.VMEM((rows_per_w, d), jnp.float32),
            pltpu.VMEM((_C, M, d), jnp.float32),
            pltpu.VMEM((_C, M, d), jnp.float32),
            pltpu.VMEM((_C, d), jnp.float32),
            pltpu.SemaphoreType.DMA,
            pltpu.SemaphoreType.DMA,
        ],
    )
    def sc_attend(h_hbm, mem_hbm, r_hbm, h_v, mb0, mb1, r_v, sem0, sem1):
        wid = lax.axis_index("s") * _NC + lax.axis_index("c")
        out_base = wid * rows_per_w
        base = row0 + out_base
        pltpu.sync_copy(h_hbm.at[pl.ds(base, rows_per_w)], h_v)

        def compute_chunk(chunk_idx, mb):
            def row_body(c, _):
                hv = [h_v[chunk_idx * _C + c, pl.ds(k * _L, _L)]
                      for k in range(nk)]
                den0 = jnp.zeros((_L,), jnp.float32)
                racc0 = [jnp.zeros((_L,), jnp.float32) for _ in range(nk)]

                def m_body(i, carry):
                    den = carry[0]
                    racc = list(carry[1:])
                    for u in range(2):
                        m = 2 * i + u
                        mv = [mb[c, m, pl.ds(k * _L, _L)] for k in range(nk)]
                        s = mv[0] * hv[0]
                        for k in range(1, nk):
                            s = s + mv[k] * hv[k]
                        e = jnp.exp(_butterfly_sum(s))
                        den = den + e
                        for k in range(nk):
                            racc[k] = racc[k] + e * mv[k]
                    return tuple([den] + racc)

                out = lax.fori_loop(0, M // 2, m_body, tuple([den0] + racc0))
                den = out[0]
                inv = 1.0 / den
                for k in range(nk):
                    r_v[c, pl.ds(k * _L, _L)] = out[1 + k] * inv
                return 0

            lax.fori_loop(0, _C, row_body, 0)
            pltpu.sync_copy(
                r_v, r_hbm.at[pl.ds(out_base + chunk_idx * _C, _C)])

        cp0 = pltpu.async_copy(mem_hbm.at[pl.ds(base, _C)], mb0, sem0)

        def outer(t, _):
            ca = 2 * t
            row_a = base + ca * _C
            cpb = pltpu.async_copy(
                mem_hbm.at[pl.ds(row_a + _C, _C)], mb1, sem1)
            pltpu.make_async_copy(
                mem_hbm.at[pl.ds(row_a, _C)], mb0, sem0).wait()
            compute_chunk(ca, mb0)

            @pl.when(t < n_chunks // 2 - 1)
            def _():
                pltpu.async_copy(
                    mem_hbm.at[pl.ds(row_a + 2 * _C, _C)], mb0, sem0)

            cpb.wait()
            compute_chunk(ca + 1, mb1)
            return 0

        lax.fori_loop(0, n_chunks // 2, outer, 0)

    return sc_attend


_R = 512
_RT = 256
_B_TC = 2048


def _gate_body(h_ref, r_ref, wg_ref, ug_ref, bias_ref, out_ref):
    h = h_ref[...]
    r = r_ref[...]
    z = jnp.dot(h, wg_ref[...], preferred_element_type=jnp.float32)
    z = z + jnp.dot(r, ug_ref[...], preferred_element_type=jnp.float32)
    g = jax.nn.sigmoid(z + bias_ref[...])
    out_ref[...] = g * r + (1.0 - g) * h


def _tc_attend_body(h_ref, mem_ref, out_ref):
    h = h_ref[...]
    mem = mem_ref[...]
    scores = jnp.sum(mem * h[:, None, :], axis=2)
    e = jnp.exp(scores)
    attn = e / jnp.sum(e, axis=1, keepdims=True)
    out_ref[...] = jnp.sum(attn[:, :, None] * mem, axis=1)


def _tc_attend(h, mem, nrows):
    B, M, d = mem.shape
    return pl.pallas_call(
        _tc_attend_body,
        grid=(nrows // _RT,),
        in_specs=[
            pl.BlockSpec((_RT, d), lambda i: (i, 0)),
            pl.BlockSpec((_RT, M, d), lambda i: (i, 0, 0)),
        ],
        out_specs=pl.BlockSpec((_RT, d), lambda i: (i, 0)),
        out_shape=jax.ShapeDtypeStruct((nrows, d), jnp.float32),
        compiler_params=pltpu.CompilerParams(
            dimension_semantics=("parallel",),
        ),
    )(h, mem)


@jax.jit
def kernel(h_tilde, mem_bank, W_g_w, W_g_b, U_g_w, U_g_b, b_g):
    B, M, d = mem_bank.shape
    r_sc = _make_sc_attend(B, M, d, _B_TC, B - _B_TC)(h_tilde, mem_bank)
    r_tc = _tc_attend(h_tilde, mem_bank, _B_TC)
    r = jnp.concatenate([r_tc, r_sc], axis=0)
    wg = W_g_w.T
    ug = U_g_w.T
    bias = (W_g_b + U_g_b + b_g).reshape(1, d)
    return pl.pallas_call(
        _gate_body,
        grid=(B // _R,),
        in_specs=[
            pl.BlockSpec((_R, d), lambda i: (i, 0)),
            pl.BlockSpec((_R, d), lambda i: (i, 0)),
            pl.BlockSpec((d, d), lambda i: (0, 0)),
            pl.BlockSpec((d, d), lambda i: (0, 0)),
            pl.BlockSpec((1, d), lambda i: (0, 0)),
        ],
        out_specs=pl.BlockSpec((_R, d), lambda i: (i, 0)),
        out_shape=jax.ShapeDtypeStruct((B, d), jnp.float32),
        compiler_params=pltpu.CompilerParams(
            dimension_semantics=("arbitrary",),
        ),
    )(h_tilde, r, wg, ug, bias)


# speedup vs baseline: 1.0894x; 1.0894x over previous
import functools
import jax
import jax.numpy as jnp
from jax import lax
from jax.experimental import pallas as pl
from jax.experimental.pallas import tpu as pltpu
from jax.experimental.pallas import tpu_sc as plsc

_NC = 2
_NS = 16
_L = 16
_C = 4


def _lane_gather(v, idx):
    dnums = lax.GatherDimensionNumbers(
        offset_dims=(), collapsed_slice_dims=(0,), start_index_map=(0,))
    return lax.gather(
        v, idx[:, None], dnums, slice_sizes=(1,),
        mode=lax.GatherScatterMode.PROMISE_IN_BOUNDS)


def _butterfly_sum(v):
    lanes = jnp.arange(_L, dtype=jnp.int32)
    for s in (1, 2, 4, 8):
        v = v + _lane_gather(v, lanes ^ s)
    return v


def _make_sc_attend(B, M, d, row0, nrows):
    nw = _NC * _NS
    rows_per_w = nrows // nw
    n_chunks = rows_per_w // _C
    nk = d // _L
    mesh = plsc.VectorSubcoreMesh(core_axis_name="c", subcore_axis_name="s")

    @functools.partial(
        pl.kernel,
        mesh=mesh,
        out_type=jax.ShapeDtypeStruct((nrows, d), jnp.float32),
        scratch_types=[
            pltpu.VMEM((rows_per_w, d), jnp.float32),
            pltpu.VMEM((_C, M, d), jnp.float32),
            pltpu.VMEM((_C, M, d), jnp.float32),
            pltpu.VMEM((_C, d), jnp.float32),
            pltpu.SemaphoreType.DMA,
            pltpu.SemaphoreType.DMA,
        ],
    )
    def sc_attend(h_hbm, mem_hbm, r_hbm, h_v, mb0, mb1, r_v, sem0, sem1):
        wid = lax.axis_index("s") * _NC + lax.axis_index("c")
        out_base = wid * rows_per_w
        base = row0 + out_base
        pltpu.sync_copy(h_hbm.at[pl.ds(base, rows_per_w)], h_v)

        def compute_chunk(chunk_idx, mb):
            def row_body(c, _):
                hv = [h_v[chunk_idx * _C + c, pl.ds(k * _L, _L)]
                      for k in range(nk)]
                den0 = jnp.zeros((_L,), jnp.float32)
                racc0 = [jnp.zeros((_L,), jnp.float32) for _ in range(nk)]

                def m_body(m, carry):
                    den = carry[0]
                    racc = list(carry[1:])
                    mv = [mb[c, m, pl.ds(k * _L, _L)] for k in range(nk)]
                    s = mv[0] * hv[0]
                    for k in range(1, nk):
                        s = s + mv[k] * hv[k]
                    e = jnp.exp(_butterfly_sum(s))
                    den = den + e
                    for k in range(nk):
                        racc[k] = racc[k] + e * mv[k]
                    return tuple([den] + racc)

                out = lax.fori_loop(0, M, m_body, tuple([den0] + racc0))
                den = out[0]
                inv = 1.0 / den
                for k in range(nk):
                    r_v[c, pl.ds(k * _L, _L)] = out[1 + k] * inv
                return 0

            lax.fori_loop(0, _C, row_body, 0)
            pltpu.sync_copy(
                r_v, r_hbm.at[pl.ds(out_base + chunk_idx * _C, _C)])

        cp0 = pltpu.async_copy(mem_hbm.at[pl.ds(base, _C)], mb0, sem0)

        def outer(t, _):
            ca = 2 * t
            row_a = base + ca * _C
            cpb = pltpu.async_copy(
                mem_hbm.at[pl.ds(row_a + _C, _C)], mb1, sem1)
            pltpu.make_async_copy(
                mem_hbm.at[pl.ds(row_a, _C)], mb0, sem0).wait()
            compute_chunk(ca, mb0)

            @pl.when(t < n_chunks // 2 - 1)
            def _():
                pltpu.async_copy(
                    mem_hbm.at[pl.ds(row_a + 2 * _C, _C)], mb0, sem0)

            cpb.wait()
            compute_chunk(ca + 1, mb1)
            return 0

        lax.fori_loop(0, n_chunks // 2, outer, 0)

    return sc_attend


_R = 512
_RT = 256
_B_TC = 2048


def _gate_body(h_ref, r_ref, wg_ref, ug_ref, bias_ref, out_ref):
    h = h_ref[...]
    r = r_ref[...]
    z = jnp.dot(h, wg_ref[...], preferred_element_type=jnp.float32)
    z = z + jnp.dot(r, ug_ref[...], preferred_element_type=jnp.float32)
    g = jax.nn.sigmoid(z + bias_ref[...])
    out_ref[...] = g * r + (1.0 - g) * h


def _tc_fused_body(h_ref, mem_ref, wg_ref, ug_ref, bias_ref, out_ref):
    h = h_ref[...]
    mem = mem_ref[...]
    scores = jnp.sum(mem * h[:, None, :], axis=2)
    e = jnp.exp(scores)
    attn = e / jnp.sum(e, axis=1, keepdims=True)
    r = jnp.sum(attn[:, :, None] * mem, axis=1)
    z = jnp.dot(h, wg_ref[...], preferred_element_type=jnp.float32)
    z = z + jnp.dot(r, ug_ref[...], preferred_element_type=jnp.float32)
    g = jax.nn.sigmoid(z + bias_ref[...])
    out_ref[...] = g * r + (1.0 - g) * h


def _tc_fused(h, mem, wg, ug, bias, nrows):
    B, M, d = mem.shape
    return pl.pallas_call(
        _tc_fused_body,
        grid=(nrows // _RT,),
        in_specs=[
            pl.BlockSpec((_RT, d), lambda i: (i, 0)),
            pl.BlockSpec((_RT, M, d), lambda i: (i, 0, 0)),
            pl.BlockSpec((d, d), lambda i: (0, 0)),
            pl.BlockSpec((d, d), lambda i: (0, 0)),
            pl.BlockSpec((1, d), lambda i: (0, 0)),
        ],
        out_specs=pl.BlockSpec((_RT, d), lambda i: (i, 0)),
        out_shape=jax.ShapeDtypeStruct((nrows, d), jnp.float32),
        compiler_params=pltpu.CompilerParams(
            dimension_semantics=("arbitrary",),
        ),
    )(h, mem, wg, ug, bias)


@jax.jit
def kernel(h_tilde, mem_bank, W_g_w, W_g_b, U_g_w, U_g_b, b_g):
    B, M, d = mem_bank.shape
    wg = W_g_w.T
    ug = U_g_w.T
    bias = (W_g_b + U_g_b + b_g).reshape(1, d)
    r_sc = _make_sc_attend(B, M, d, _B_TC, B - _B_TC)(h_tilde, mem_bank)
    u_tc = _tc_fused(h_tilde, mem_bank, wg, ug, bias, _B_TC)
    n_sc = B - _B_TC
    off = _B_TC // _R
    u_sc = pl.pallas_call(
        _gate_body,
        grid=(n_sc // _R,),
        in_specs=[
            pl.BlockSpec((_R, d), lambda i: (i + off, 0)),
            pl.BlockSpec((_R, d), lambda i: (i, 0)),
            pl.BlockSpec((d, d), lambda i: (0, 0)),
            pl.BlockSpec((d, d), lambda i: (0, 0)),
            pl.BlockSpec((1, d), lambda i: (0, 0)),
        ],
        out_specs=pl.BlockSpec((_R, d), lambda i: (i, 0)),
        out_shape=jax.ShapeDtypeStruct((n_sc, d), jnp.float32),
        compiler_params=pltpu.CompilerParams(
            dimension_semantics=("arbitrary",),
        ),
    )(h_tilde, r_sc, wg, ug, bias)
    return jnp.concatenate([u_tc, u_sc], axis=0)
